# baseline (speedup 1.0000x reference)
import jax
import jax.numpy as jnp
from jax.experimental import pallas as pl
from jax.experimental.pallas import tpu as pltpu


def _gru_layer(xseq, Wih, Whh, bih, bhh):
    def step(hprev, xt):
        gi = xt @ Wih.T + bih
        gh = hprev @ Whh.T + bhh
        ir, iz, inn = jnp.split(gi, 3, axis=-1)
        hr, hz, hn = jnp.split(gh, 3, axis=-1)
        r = jax.nn.sigmoid(ir + hr)
        z = jax.nn.sigmoid(iz + hz)
        ncand = jnp.tanh(inn + r * hn)
        hnew = (1.0 - z) * ncand + z * hprev
        return hnew, hnew
    h0 = jnp.zeros((xseq.shape[0], Whh.shape[1]), dtype=jnp.float32)
    _, ys = jax.lax.scan(step, h0, jnp.swapaxes(xseq, 0, 1))
    return jnp.swapaxes(ys, 0, 1)


def _mlp_kernel(att_ref, w1_ref, b1_ref, w2_ref, b2_ref, o_ref):
    hh = jax.nn.relu(att_ref[...] @ w1_ref[...] + b1_ref[...])
    o_ref[...] = hh @ w2_ref[...] + b2_ref[...]


def kernel(x, gcn_w1, gcn_b1, gcn_w2, gcn_b2, gru_wih0, gru_whh0, gru_bih0, gru_bhh0, gru_wih1, gru_whh1, gru_bih1, gru_bhh1, att_w1, att_b1, att_w2, out_w1, out_b1, out_w2, out_b2, edge_index):
    n = x.shape[0]
    loop = jnp.arange(n, dtype=edge_index.dtype)
    src = jnp.concatenate([edge_index[0], loop])
    dst = jnp.concatenate([edge_index[1], loop])
    deg = jnp.zeros((n,), jnp.float32).at[dst].add(1.0)
    dinv = jnp.where(deg > 0.0, jax.lax.rsqrt(deg), 0.0)
    norm = dinv[src] * dinv[dst]

    def gcn(h, W, b):
        hw = h @ W
        m = hw[src] * norm[:, None]
        agg = jnp.zeros((n, hw.shape[1]), jnp.float32).at[dst].add(m)
        return agg + b

    embs = []
    for t in range(x.shape[1]):
        h = x[:, t, :]
        h = jax.nn.relu(gcn(h, gcn_w1, gcn_b1))
        h = jax.nn.relu(gcn(h, gcn_w2, gcn_b2))
        embs.append(h)
    seq = jnp.stack(embs, axis=1)
    r1 = _gru_layer(seq, gru_wih0, gru_whh0, gru_bih0, gru_bhh0)
    r2 = _gru_layer(r1, gru_wih1, gru_whh1, gru_bih1, gru_bhh1)
    scores = (jnp.tanh(r2 @ att_w1 + att_b1) @ att_w2)[..., 0]
    w = jax.nn.softmax(scores, axis=-1)
    attended = jnp.einsum('nt,nth->nh', w, r2)

    NP = 50176
    attended = jnp.pad(attended, ((0, NP - n), (0, 0)))
    BN = 1024
    out = pl.pallas_call(
        _mlp_kernel,
        grid=(NP // BN,),
        in_specs=[
            pl.BlockSpec((BN, 64), lambda i: (i, 0)),
            pl.BlockSpec((64, 32), lambda i: (0, 0)),
            pl.BlockSpec((32,), lambda i: (0,)),
            pl.BlockSpec((32, 1), lambda i: (0, 0)),
            pl.BlockSpec((1,), lambda i: (0,)),
        ],
        out_specs=pl.BlockSpec((BN, 1), lambda i: (i, 0)),
        out_shape=jax.ShapeDtypeStruct((NP, 1), jnp.float32),
    )(attended, out_w1, out_b1, out_w2, out_b2)
    return out[:n]
